# baseline (speedup 1.0000x reference)
import functools

import jax
import jax.numpy as jnp
from jax import lax
from jax.experimental import pallas as pl
from jax.experimental.pallas import tpu as pltpu
from jax.experimental.pallas import tpu_sc as plsc

N = 10000
D = 128
E = 320000
E_TOT = E + N
NC, NS, L = 2, 16, 16
NW = NC * NS
CHUNK = 128
RPT = 84
PAD_E = NW * RPT * CHUNK
NP = 10112
ROWS_PER_SUB = NP // NS


def _tc_feat_body(x_ref, w_ref, att2_ref, h_ref, a2_ref):
    h = lax.dot_general(x_ref[...], w_ref[...], (((1,), (1,)), ((), ())),
                        preferred_element_type=jnp.float32)
    h_ref[...] = h
    a2_ref[...] = lax.dot_general(h, att2_ref[...], (((1,), (0,)), ((), ())),
                                  preferred_element_type=jnp.float32)


def _tc_feat(x_pad, w, att2):
    return pl.pallas_call(
        _tc_feat_body,
        out_shape=(jax.ShapeDtypeStruct((NP, D), jnp.float32),
                   jax.ShapeDtypeStruct((NP, 2), jnp.float32)),
    )(x_pad, w, att2)


def _tc_mid_body(num_ref, den_ref, b_ref, w_ref, att2_ref, h_ref, a2_ref):
    den = den_ref[0, :] + den_ref[1, :]
    o = (num_ref[0] + num_ref[1]) / (den[:, None] + 1e-16) + b_ref[...]
    o = jnp.where(o > 0, o, 0.02 * o)
    h = lax.dot_general(o, w_ref[...], (((1,), (1,)), ((), ())),
                        preferred_element_type=jnp.float32)
    h_ref[...] = h
    a2_ref[...] = lax.dot_general(h, att2_ref[...], (((1,), (0,)), ((), ())),
                                  preferred_element_type=jnp.float32)


def _tc_mid(num, den, b, w, att2):
    return pl.pallas_call(
        _tc_mid_body,
        out_shape=(jax.ShapeDtypeStruct((NP, D), jnp.float32),
                   jax.ShapeDtypeStruct((NP, 2), jnp.float32)),
    )(num, den, b, w, att2)


def _tc_fin_body(num_ref, den_ref, b_ref, h_ref):
    den = den_ref[0, :] + den_ref[1, :]
    o = (num_ref[0] + num_ref[1]) / (den[:, None] + 1e-16) + b_ref[...]
    o = jnp.where(o > 0, o, 0.02 * o)
    h_ref[...] = o[:N, :]


def _tc_fin(num, den, b):
    return pl.pallas_call(
        _tc_fin_body,
        out_shape=jax.ShapeDtypeStruct((N, D), jnp.float32),
    )(num, den, b)


BM = 400


def _tc_mm_body(a_ref, b_ref, o_ref):
    o_ref[...] = lax.dot_general(a_ref[...], b_ref[...],
                                 (((1,), (1,)), ((), ())),
                                 preferred_element_type=jnp.float32)


def _tc_mm(h):
    return pl.pallas_call(
        _tc_mm_body,
        grid=(N // BM,),
        in_specs=[pl.BlockSpec((BM, D), lambda i: (i, 0)),
                  pl.BlockSpec((N, D), lambda i: (0, 0))],
        out_specs=pl.BlockSpec((BM, N), lambda i: (i, 0)),
        out_shape=jax.ShapeDtypeStruct((N, N), jnp.float32),
    )(h, h)


_MESH = plsc.VectorSubcoreMesh(core_axis_name="c", subcore_axis_name="s",
                               num_cores=NC, num_subcores=NS)


@functools.partial(
    pl.kernel,
    out_type=(jax.ShapeDtypeStruct((NC, NP, D), jnp.float32),
              jax.ShapeDtypeStruct((NC * NP,), jnp.float32)),
    mesh=_MESH,
    compiler_params=pltpu.CompilerParams(needs_layout_passes=False),
    scratch_types=[
        pltpu.VMEM((2, CHUNK), jnp.int32),
        pltpu.VMEM((2, CHUNK), jnp.int32),
        pltpu.VMEM((2, CHUNK), jnp.int32),
        pltpu.VMEM((2, CHUNK), jnp.int32),
        pltpu.VMEM((CHUNK, D), jnp.float32),
        pltpu.VMEM((CHUNK, D), jnp.float32),
        pltpu.VMEM((CHUNK,), jnp.float32),
        pltpu.VMEM((CHUNK,), jnp.float32),
        pltpu.VMEM((CHUNK,), jnp.float32),
        pltpu.VMEM((CHUNK,), jnp.float32),
        pltpu.VMEM((CHUNK + L,), jnp.float32),
        pltpu.VMEM((CHUNK + L,), jnp.float32),
        pltpu.VMEM((640,), jnp.float32),
        pltpu.MemorySpace.VMEM_SHARED((NP, D), jnp.float32),
        pltpu.MemorySpace.VMEM_SHARED((NP,), jnp.float32),
        pltpu.SemaphoreType.DMA,
        pltpu.SemaphoreType.DMA,
        pltpu.SemaphoreType.DMA,
        pltpu.SemaphoreType.DMA,
        pltpu.SemaphoreType.DMA,
        pltpu.SemaphoreType.DMA,
        pltpu.SemaphoreType.DMA,
        pltpu.SemaphoreType.DMA,
    ],
)
def _sc_edge(sd_hbm, asrc_hbm, adst_hbm, h_hbm, num_out, den_out,
             sdv0, sdv1, sdv2, sdv3, rows0, rows1, wa0, wa1, wb0, wb1,
             wv0, wv1, zv, num_sh, den_sh,
             semI0, semI1, semI2, semI3, semG0, semG1, semS0, semS1):
    SDV = [sdv0, sdv1, sdv2, sdv3]
    ROWS = [rows0, rows1]
    WA = [wa0, wa1]
    WB = [wb0, wb1]
    WV = [wv0, wv1]
    SEMI = [semI0, semI1, semI2, semI3]
    SEMG = [semG0, semG1]
    SEMS = [semS0, semS1]

    c = lax.axis_index("c")
    s = lax.axis_index("s")
    wid = c * NS + s

    zero16 = jnp.zeros((L,), jnp.float32)

    def _z(i, _):
        zv[pl.ds(i * L, L)] = zero16
        return ()
    lax.fori_loop(0, 640 // L, _z, ())
    base = s * ROWS_PER_SUB
    pltpu.sync_copy(zv.at[pl.ds(0, ROWS_PER_SUB)],
                    den_sh.at[pl.ds(base, ROWS_PER_SUB)])

    def _zrows(i, _):
        def _zcol(u, _):
            rows0[i, pl.ds(u * L, L)] = zero16
            return ()
        lax.fori_loop(0, D // L, _zcol, ())
        return ()
    lax.fori_loop(0, CHUNK, _zrows, ())
    for k in range(5):
        sz = min(CHUNK, ROWS_PER_SUB - k * CHUNK)
        pltpu.sync_copy(rows0.at[pl.ds(0, sz)],
                        num_sh.at[pl.ds(base + k * CHUNK, sz)])
    plsc.subcore_barrier()

    pltpu.async_copy(sd_hbm.at[wid, 0], sdv0, semI0)
    pltpu.async_copy(sd_hbm.at[wid, 1], sdv1, semI1)
    pltpu.make_async_copy(sd_hbm.at[wid, 0], sdv0, semI0).wait()
    pltpu.async_copy(h_hbm.at[sdv0.at[0]], rows0, semG0)
    pltpu.async_copy(asrc_hbm.at[sdv0.at[0]], wa0, semG0)
    pltpu.async_copy(adst_hbm.at[sdv0.at[1]], wb0, semG0)

    def _super(jj, _):
        for q in range(4):
            k = jj * 4 + q
            b = q % 2
            b1 = (q + 1) % 2
            qi = q
            qn = (q + 1) % 4
            qp = (q + 3) % 4
            qf = (q + 2) % 4
            pltpu.make_async_copy(h_hbm.at[SDV[qi].at[0]], ROWS[b],
                                  SEMG[b]).wait()
            pltpu.make_async_copy(asrc_hbm.at[SDV[qi].at[0]], WA[b],
                                  SEMG[b]).wait()
            pltpu.make_async_copy(adst_hbm.at[SDV[qi].at[1]], WB[b],
                                  SEMG[b]).wait()
            for v in range(CHUNK // L):
                a = WA[b][pl.ds(v * L, L)] + WB[b][pl.ds(v * L, L)]
                e = jnp.where(a > 0, a, 0.2 * a)
                WV[b][pl.ds(L + v * L, L)] = jnp.exp(e)
            wvb = WV[b]
            rowsb = ROWS[b]

            def _scale_grp(g, _):
                rg = g * 8
                for r in range(8):
                    wr_idx = jnp.zeros((L,), jnp.int32) + (rg + (L + r))
                    wr = plsc.load_gather(wvb, [wr_idx])
                    for u in range(D // L):
                        rowsb[rg + r, pl.ds(u * L, L)] = (
                            rowsb[rg + r, pl.ds(u * L, L)] * wr)
                return ()
            lax.fori_loop(0, CHUNK // 8, _scale_grp, ())
            @pl.when(k >= 1)
            def _():
                pltpu.make_async_copy(ROWS[b1], num_sh.at[SDV[qp].at[1]],
                                      SEMS[b1]).wait()
                pltpu.make_async_copy(WV[b1].at[pl.ds(L, CHUNK)],
                                      den_sh.at[SDV[qp].at[1]],
                                      SEMS[b1]).wait()
            pltpu.async_copy(ROWS[b], num_sh.at[SDV[qi].at[1]], SEMS[b],
                             add=True)
            pltpu.async_copy(WV[b].at[pl.ds(L, CHUNK)],
                             den_sh.at[SDV[qi].at[1]], SEMS[b], add=True)
            @pl.when(k + 2 < RPT)
            def _():
                pltpu.async_copy(sd_hbm.at[wid, k + 2], SDV[qf], SEMI[qf])
            @pl.when(k + 1 < RPT)
            def _():
                pltpu.make_async_copy(sd_hbm.at[wid, k + 1], SDV[qn],
                                      SEMI[qn]).wait()
                pltpu.async_copy(h_hbm.at[SDV[qn].at[0]], ROWS[b1], SEMG[b1])
                pltpu.async_copy(asrc_hbm.at[SDV[qn].at[0]], WA[b1], SEMG[b1])
                pltpu.async_copy(adst_hbm.at[SDV[qn].at[1]], WB[b1], SEMG[b1])
        return ()

    lax.fori_loop(0, RPT // 4, _super, ())

    pltpu.make_async_copy(rows1, num_sh.at[sdv3.at[1]], semS1).wait()
    pltpu.make_async_copy(wv1.at[pl.ds(L, CHUNK)], den_sh.at[sdv3.at[1]],
                          semS1).wait()
    plsc.subcore_barrier()

    pltpu.sync_copy(num_sh.at[pl.ds(base, ROWS_PER_SUB)],
                    num_out.at[c, pl.ds(base, ROWS_PER_SUB)])
    pltpu.sync_copy(den_sh.at[pl.ds(base, ROWS_PER_SUB)],
                    zv.at[pl.ds(0, ROWS_PER_SUB)])
    pltpu.sync_copy(zv.at[pl.ds(0, ROWS_PER_SUB)],
                    den_out.at[pl.ds(c * NP + base, ROWS_PER_SUB)])


def kernel(x, edge_index, W1, att_src1, att_dst1, b1, W2, att_src2,
           att_dst2, b2):
    loop = jnp.arange(N, dtype=jnp.int32)
    pad = PAD_E - E_TOT
    src = jnp.concatenate([edge_index[0], loop,
                           jnp.full((pad,), N, jnp.int32)])
    dst = jnp.concatenate([edge_index[1], loop,
                           jnp.zeros((pad,), jnp.int32)])
    sd = jnp.stack([src.reshape(NW, RPT, CHUNK),
                    dst.reshape(NW, RPT, CHUNK)], axis=2)
    x_pad = jnp.zeros((NP, D), jnp.float32).at[:N].set(x)
    att2_1 = jnp.stack([att_src1, att_dst1], axis=1)
    att2_2 = jnp.stack([att_src2, att_dst2], axis=1)

    h1, a2_1 = _tc_feat(x_pad, W1, att2_1)
    asrc1 = a2_1[:, 0].at[N:].set(-1e30)
    adst1 = a2_1[:, 1]
    num1, den1 = _sc_edge(sd, asrc1, adst1, h1)
    den1 = den1.reshape(NC, NP)

    h2, a2_2 = _tc_mid(num1, den1, b1.reshape(1, D), W2, att2_2)
    asrc2 = a2_2[:, 0].at[N:].set(-1e30)
    adst2 = a2_2[:, 1]
    num2, den2 = _sc_edge(sd, asrc2, adst2, h2)
    den2 = den2.reshape(NC, NP)

    h2b = _tc_fin(num2, den2, b2.reshape(1, D))
    return _tc_mm(h2b)


# speedup vs baseline: 2.3894x; 2.3894x over previous
import functools

import jax
import jax.numpy as jnp
from jax import lax
from jax.experimental import pallas as pl
from jax.experimental.pallas import tpu as pltpu
from jax.experimental.pallas import tpu_sc as plsc

N = 10000
D = 128
E = 320000
E_TOT = E + N
NC, NS, L = 2, 16, 16
NW = NC * NS
CHUNK = 96
RPT = 108
PAD_E = NW * RPT * CHUNK
NP = 10112
ROWS_PER_SUB = NP // NS


def _tc_feat_body(x_ref, w_ref, att2_ref, h_ref, a2_ref):
    h = lax.dot_general(x_ref[...], w_ref[...], (((1,), (1,)), ((), ())),
                        preferred_element_type=jnp.float32)
    h_ref[...] = h
    a2_ref[...] = lax.dot_general(h, att2_ref[...], (((1,), (0,)), ((), ())),
                                  preferred_element_type=jnp.float32)


def _tc_feat(x_pad, w, att2):
    return pl.pallas_call(
        _tc_feat_body,
        out_shape=(jax.ShapeDtypeStruct((NP, D), jnp.float32),
                   jax.ShapeDtypeStruct((NP, 2), jnp.float32)),
    )(x_pad, w, att2)


def _tc_mid_body(num_ref, den_ref, b_ref, w_ref, att2_ref, h_ref, a2_ref):
    den = den_ref[0, :] + den_ref[1, :]
    o = (num_ref[0] + num_ref[1]) / (den[:, None] + 1e-16) + b_ref[...]
    o = jnp.where(o > 0, o, 0.02 * o)
    h = lax.dot_general(o, w_ref[...], (((1,), (1,)), ((), ())),
                        preferred_element_type=jnp.float32)
    h_ref[...] = h
    a2_ref[...] = lax.dot_general(h, att2_ref[...], (((1,), (0,)), ((), ())),
                                  preferred_element_type=jnp.float32)


def _tc_mid(num, den, b, w, att2):
    return pl.pallas_call(
        _tc_mid_body,
        out_shape=(jax.ShapeDtypeStruct((NP, D), jnp.float32),
                   jax.ShapeDtypeStruct((NP, 2), jnp.float32)),
    )(num, den, b, w, att2)


def _tc_fin_body(num_ref, den_ref, b_ref, h_ref):
    den = den_ref[0, :] + den_ref[1, :]
    o = (num_ref[0] + num_ref[1]) / (den[:, None] + 1e-16) + b_ref[...]
    o = jnp.where(o > 0, o, 0.02 * o)
    h_ref[...] = o[:N, :]


def _tc_fin(num, den, b):
    return pl.pallas_call(
        _tc_fin_body,
        out_shape=jax.ShapeDtypeStruct((N, D), jnp.float32),
    )(num, den, b)


BM = 400


def _tc_mm_body(a_ref, b_ref, o_ref):
    o_ref[...] = lax.dot_general(a_ref[...], b_ref[...],
                                 (((1,), (1,)), ((), ())),
                                 preferred_element_type=jnp.float32)


def _tc_mm(h):
    return pl.pallas_call(
        _tc_mm_body,
        grid=(N // BM,),
        in_specs=[pl.BlockSpec((BM, D), lambda i: (i, 0)),
                  pl.BlockSpec((N, D), lambda i: (0, 0))],
        out_specs=pl.BlockSpec((BM, N), lambda i: (i, 0)),
        out_shape=jax.ShapeDtypeStruct((N, N), jnp.float32),
    )(h, h)


_MESH = plsc.VectorSubcoreMesh(core_axis_name="c", subcore_axis_name="s",
                               num_cores=NC, num_subcores=NS)


@functools.partial(
    pl.kernel,
    out_type=(jax.ShapeDtypeStruct((NC, NP, D), jnp.float32),
              jax.ShapeDtypeStruct((NC * NP,), jnp.float32)),
    mesh=_MESH,
    compiler_params=pltpu.CompilerParams(needs_layout_passes=False),
    scratch_types=[
        pltpu.VMEM((2, CHUNK), jnp.int32),
        pltpu.VMEM((2, CHUNK), jnp.int32),
        pltpu.VMEM((2, CHUNK), jnp.int32),
        pltpu.VMEM((2, CHUNK), jnp.int32),
        pltpu.VMEM((CHUNK, D), jnp.float32),
        pltpu.VMEM((CHUNK, D), jnp.float32),
        pltpu.VMEM((NP,), jnp.float32),
        pltpu.VMEM((NP,), jnp.float32),
        pltpu.VMEM((CHUNK + L,), jnp.float32),
        pltpu.VMEM((CHUNK + L,), jnp.float32),
        pltpu.VMEM((640,), jnp.float32),
        pltpu.MemorySpace.VMEM_SHARED((NP, D), jnp.float32),
        pltpu.MemorySpace.VMEM_SHARED((NP,), jnp.float32),
        pltpu.SemaphoreType.DMA,
        pltpu.SemaphoreType.DMA,
        pltpu.SemaphoreType.DMA,
        pltpu.SemaphoreType.DMA,
        pltpu.SemaphoreType.DMA,
        pltpu.SemaphoreType.DMA,
        pltpu.SemaphoreType.DMA,
        pltpu.SemaphoreType.DMA,
    ],
)
def _sc_edge(sd_hbm, asrc_hbm, adst_hbm, h_hbm, num_out, den_out,
             sdv0, sdv1, sdv2, sdv3, rows0, rows1, asv, adv,
             wv0, wv1, zv, num_sh, den_sh,
             semI0, semI1, semI2, semI3, semG0, semG1, semS0, semS1):
    SDV = [sdv0, sdv1, sdv2, sdv3]
    ROWS = [rows0, rows1]
    WV = [wv0, wv1]
    SEMI = [semI0, semI1, semI2, semI3]
    SEMG = [semG0, semG1]
    SEMS = [semS0, semS1]

    c = lax.axis_index("c")
    s = lax.axis_index("s")
    wid = c * NS + s

    zero16 = jnp.zeros((L,), jnp.float32)

    def _z(i, _):
        zv[pl.ds(i * L, L)] = zero16
        return ()
    lax.fori_loop(0, 640 // L, _z, ())
    base = s * ROWS_PER_SUB
    pltpu.sync_copy(zv.at[pl.ds(0, ROWS_PER_SUB)],
                    den_sh.at[pl.ds(base, ROWS_PER_SUB)])

    def _zrows(i, _):
        def _zcol(u, _):
            rows0[i, pl.ds(u * L, L)] = zero16
            return ()
        lax.fori_loop(0, D // L, _zcol, ())
        return ()
    lax.fori_loop(0, CHUNK, _zrows, ())
    nz = (ROWS_PER_SUB + CHUNK - 1) // CHUNK
    for k in range(nz):
        sz = min(CHUNK, ROWS_PER_SUB - k * CHUNK)
        pltpu.sync_copy(rows0.at[pl.ds(0, sz)],
                        num_sh.at[pl.ds(base + k * CHUNK, sz)])
    plsc.subcore_barrier()

    pltpu.sync_copy(asrc_hbm, asv)
    pltpu.sync_copy(adst_hbm, adv)
    pltpu.async_copy(sd_hbm.at[wid, 0], sdv0, semI0)
    pltpu.async_copy(sd_hbm.at[wid, 1], sdv1, semI1)
    pltpu.make_async_copy(sd_hbm.at[wid, 0], sdv0, semI0).wait()
    pltpu.async_copy(h_hbm.at[sdv0.at[0]], rows0, semG0)

    def _super(jj, _):
        for q in range(4):
            k = jj * 4 + q
            b = q % 2
            b1 = (q + 1) % 2
            qi = q
            qn = (q + 1) % 4
            qp = (q + 3) % 4
            qf = (q + 2) % 4
            pltpu.make_async_copy(h_hbm.at[SDV[qi].at[0]], ROWS[b],
                                  SEMG[b]).wait()
            for v in range(CHUNK // L):
                si = SDV[qi][0, pl.ds(v * L, L)]
                di = SDV[qi][1, pl.ds(v * L, L)]
                a = plsc.load_gather(asv, [si]) + plsc.load_gather(adv, [di])
                e = jnp.where(a > 0, a, 0.2 * a)
                WV[b][pl.ds(L + v * L, L)] = jnp.exp(e)
            wvb = WV[b]
            rowsb = ROWS[b]

            def _scale_grp(g, _):
                rg = g * 8
                for r in range(8):
                    wr_idx = jnp.zeros((L,), jnp.int32) + (rg + (L + r))
                    wr = plsc.load_gather(wvb, [wr_idx])
                    for u in range(D // L):
                        rowsb[rg + r, pl.ds(u * L, L)] = (
                            rowsb[rg + r, pl.ds(u * L, L)] * wr)
                return ()
            lax.fori_loop(0, CHUNK // 8, _scale_grp, ())
            @pl.when(k >= 1)
            def _():
                pltpu.make_async_copy(ROWS[b1], num_sh.at[SDV[qp].at[1]],
                                      SEMS[b1]).wait()
                pltpu.make_async_copy(WV[b1].at[pl.ds(L, CHUNK)],
                                      den_sh.at[SDV[qp].at[1]],
                                      SEMS[b1]).wait()
            pltpu.async_copy(ROWS[b], num_sh.at[SDV[qi].at[1]], SEMS[b],
                             add=True)
            pltpu.async_copy(WV[b].at[pl.ds(L, CHUNK)],
                             den_sh.at[SDV[qi].at[1]], SEMS[b], add=True)
            @pl.when(k + 2 < RPT)
            def _():
                pltpu.async_copy(sd_hbm.at[wid, k + 2], SDV[qf], SEMI[qf])
            @pl.when(k + 1 < RPT)
            def _():
                pltpu.make_async_copy(sd_hbm.at[wid, k + 1], SDV[qn],
                                      SEMI[qn]).wait()
                pltpu.async_copy(h_hbm.at[SDV[qn].at[0]], ROWS[b1], SEMG[b1])
        return ()

    lax.fori_loop(0, RPT // 4, _super, ())

    pltpu.make_async_copy(rows1, num_sh.at[sdv3.at[1]], semS1).wait()
    pltpu.make_async_copy(wv1.at[pl.ds(L, CHUNK)], den_sh.at[sdv3.at[1]],
                          semS1).wait()
    plsc.subcore_barrier()

    pltpu.sync_copy(num_sh.at[pl.ds(base, ROWS_PER_SUB)],
                    num_out.at[c, pl.ds(base, ROWS_PER_SUB)])
    pltpu.sync_copy(den_sh.at[pl.ds(base, ROWS_PER_SUB)],
                    zv.at[pl.ds(0, ROWS_PER_SUB)])
    pltpu.sync_copy(zv.at[pl.ds(0, ROWS_PER_SUB)],
                    den_out.at[pl.ds(c * NP + base, ROWS_PER_SUB)])


def kernel(x, edge_index, W1, att_src1, att_dst1, b1, W2, att_src2,
           att_dst2, b2):
    loop = jnp.arange(N, dtype=jnp.int32)
    pad = PAD_E - E_TOT
    padi = jnp.arange(pad, dtype=jnp.int32)
    src = jnp.concatenate([edge_index[0], loop, padi % N])
    dst = jnp.concatenate([edge_index[1], loop, N + padi % (NP - N)])
    sd = jnp.stack([src.reshape(NW, RPT, CHUNK),
                    dst.reshape(NW, RPT, CHUNK)], axis=2)
    x_pad = jnp.zeros((NP, D), jnp.float32).at[:N].set(x)
    att2_1 = jnp.stack([att_src1, att_dst1], axis=1)
    att2_2 = jnp.stack([att_src2, att_dst2], axis=1)

    h1, a2_1 = _tc_feat(x_pad, W1, att2_1)
    asrc1 = a2_1[:, 0].at[N:].set(-1e30)
    adst1 = a2_1[:, 1].at[N:].set(-1e30)
    num1, den1 = _sc_edge(sd, asrc1, adst1, h1)
    den1 = den1.reshape(NC, NP)

    h2, a2_2 = _tc_mid(num1, den1, b1.reshape(1, D), W2, att2_2)
    asrc2 = a2_2[:, 0].at[N:].set(-1e30)
    adst2 = a2_2[:, 1].at[N:].set(-1e30)
    num2, den2 = _sc_edge(sd, asrc2, adst2, h2)
    den2 = den2.reshape(NC, NP)

    h2b = _tc_fin(num2, den2, b2.reshape(1, D))
    return _tc_mm(h2b)


# speedup vs baseline: 2.9134x; 1.2193x over previous
import functools

import jax
import jax.numpy as jnp
from jax import lax
from jax.experimental import pallas as pl
from jax.experimental.pallas import tpu as pltpu
from jax.experimental.pallas import tpu_sc as plsc

N = 10000
D = 128
E = 320000
E_TOT = E + N
NC, NS, L = 2, 16, 16
NW = NC * NS
CHUNK = 96
RPT = 108
PAD_E = NW * RPT * CHUNK
NP = 10112
ROWS_PER_SUB = NP // NS


def _tc_feat_body(x_ref, w_ref, att2_ref, h_ref, a2_ref):
    h = lax.dot_general(x_ref[...], w_ref[...], (((1,), (1,)), ((), ())),
                        preferred_element_type=jnp.float32)
    h_ref[...] = h
    a2_ref[...] = lax.dot_general(h, att2_ref[...], (((1,), (0,)), ((), ())),
                                  preferred_element_type=jnp.float32)


def _tc_feat(x_pad, w, att2):
    return pl.pallas_call(
        _tc_feat_body,
        out_shape=(jax.ShapeDtypeStruct((NP, D), jnp.float32),
                   jax.ShapeDtypeStruct((NP, 2), jnp.float32)),
    )(x_pad, w, att2)


def _tc_mid_body(num_ref, den_ref, b_ref, w_ref, att2_ref, h_ref, a2_ref):
    den = den_ref[0, :] + den_ref[1, :]
    o = (num_ref[0] + num_ref[1]) / (den[:, None] + 1e-16) + b_ref[...]
    o = jnp.where(o > 0, o, 0.02 * o)
    h = lax.dot_general(o, w_ref[...], (((1,), (1,)), ((), ())),
                        preferred_element_type=jnp.float32)
    h_ref[...] = h
    a2_ref[...] = lax.dot_general(h, att2_ref[...], (((1,), (0,)), ((), ())),
                                  preferred_element_type=jnp.float32)


def _tc_mid(num, den, b, w, att2):
    return pl.pallas_call(
        _tc_mid_body,
        out_shape=(jax.ShapeDtypeStruct((NP, D), jnp.float32),
                   jax.ShapeDtypeStruct((NP, 2), jnp.float32)),
    )(num, den, b, w, att2)


def _tc_fin_body(num_ref, den_ref, b_ref, h_ref):
    den = den_ref[0, :] + den_ref[1, :]
    o = (num_ref[0] + num_ref[1]) / (den[:, None] + 1e-16) + b_ref[...]
    o = jnp.where(o > 0, o, 0.02 * o)
    h_ref[...] = o[:N, :]


def _tc_fin(num, den, b):
    return pl.pallas_call(
        _tc_fin_body,
        out_shape=jax.ShapeDtypeStruct((N, D), jnp.float32),
    )(num, den, b)


BM = 400


def _tc_mm_body(a_ref, b_ref, o_ref):
    o_ref[...] = lax.dot_general(a_ref[...], b_ref[...],
                                 (((1,), (1,)), ((), ())),
                                 preferred_element_type=jnp.float32)


def _tc_mm(h):
    return pl.pallas_call(
        _tc_mm_body,
        grid=(N // BM,),
        in_specs=[pl.BlockSpec((BM, D), lambda i: (i, 0)),
                  pl.BlockSpec((N, D), lambda i: (0, 0))],
        out_specs=pl.BlockSpec((BM, N), lambda i: (i, 0)),
        out_shape=jax.ShapeDtypeStruct((N, N), jnp.float32),
    )(h, h)


_MESH = plsc.VectorSubcoreMesh(core_axis_name="c", subcore_axis_name="s",
                               num_cores=NC, num_subcores=NS)


@functools.partial(
    pl.kernel,
    out_type=(jax.ShapeDtypeStruct((NC, NP, D), jnp.float32),
              jax.ShapeDtypeStruct((NC * NP,), jnp.float32)),
    mesh=_MESH,
    compiler_params=pltpu.CompilerParams(needs_layout_passes=False),
    scratch_types=[
        pltpu.VMEM((2, CHUNK), jnp.int32),
        pltpu.VMEM((2, CHUNK), jnp.int32),
        pltpu.VMEM((2, CHUNK), jnp.int32),
        pltpu.VMEM((2, CHUNK), jnp.int32),
        pltpu.VMEM((CHUNK, D), jnp.float32),
        pltpu.VMEM((CHUNK, D), jnp.float32),
        pltpu.VMEM((NP,), jnp.float32),
        pltpu.VMEM((NP,), jnp.float32),
        pltpu.VMEM((CHUNK + L,), jnp.float32),
        pltpu.VMEM((CHUNK + L,), jnp.float32),
        pltpu.VMEM((640,), jnp.float32),
        pltpu.MemorySpace.VMEM_SHARED((NP, D), jnp.float32),
        pltpu.MemorySpace.VMEM_SHARED((NP,), jnp.float32),
        pltpu.SemaphoreType.DMA,
        pltpu.SemaphoreType.DMA,
        pltpu.SemaphoreType.DMA,
        pltpu.SemaphoreType.DMA,
        pltpu.SemaphoreType.DMA,
        pltpu.SemaphoreType.DMA,
        pltpu.SemaphoreType.DMA,
        pltpu.SemaphoreType.DMA,
    ],
)
def _sc_edge(sd_hbm, asrc_hbm, adst_hbm, h_hbm, num_out, den_out,
             sdv0, sdv1, sdv2, sdv3, rows0, rows1, asv, adv,
             wv0, wv1, zv, num_sh, den_sh,
             semI0, semI1, semI2, semI3, semG0, semG1, semS0, semS1):
    SDV = [sdv0, sdv1, sdv2, sdv3]
    ROWS = [rows0, rows1]
    WV = [wv0, wv1]
    SEMI = [semI0, semI1, semI2, semI3]
    SEMG = [semG0, semG1]
    SEMS = [semS0, semS1]

    c = lax.axis_index("c")
    s = lax.axis_index("s")
    wid = c * NS + s

    zero16 = jnp.zeros((L,), jnp.float32)

    def _z(i, _):
        zv[pl.ds(i * L, L)] = zero16
        return ()
    lax.fori_loop(0, 640 // L, _z, ())
    base = s * ROWS_PER_SUB
    pltpu.sync_copy(zv.at[pl.ds(0, ROWS_PER_SUB)],
                    den_sh.at[pl.ds(base, ROWS_PER_SUB)])

    def _zrows(i, _):
        def _zcol(u, _):
            rows0[i, pl.ds(u * L, L)] = zero16
            return ()
        lax.fori_loop(0, D // L, _zcol, ())
        return ()
    lax.fori_loop(0, CHUNK, _zrows, ())
    nz = (ROWS_PER_SUB + CHUNK - 1) // CHUNK
    for k in range(nz):
        sz = min(CHUNK, ROWS_PER_SUB - k * CHUNK)
        pltpu.sync_copy(rows0.at[pl.ds(0, sz)],
                        num_sh.at[pl.ds(base + k * CHUNK, sz)])
    plsc.subcore_barrier()

    pltpu.sync_copy(asrc_hbm, asv)
    pltpu.sync_copy(adst_hbm, adv)
    pltpu.async_copy(sd_hbm.at[wid, 0], sdv0, semI0)
    pltpu.async_copy(sd_hbm.at[wid, 1], sdv1, semI1)
    pltpu.make_async_copy(sd_hbm.at[wid, 0], sdv0, semI0).wait()
    pltpu.async_copy(h_hbm.at[sdv0.at[0]], rows0, semG0)

    def _super(jj, _):
        for q in range(4):
            k = jj * 4 + q
            b = q % 2
            b1 = (q + 1) % 2
            qi = q
            qn = (q + 1) % 4
            qp = (q + 3) % 4
            qf = (q + 2) % 4
            pltpu.make_async_copy(h_hbm.at[SDV[qi].at[0]], ROWS[b],
                                  SEMG[b]).wait()
            @pl.when(k >= 1)
            def _():
                pltpu.make_async_copy(ROWS[b1], num_sh.at[SDV[qp].at[1]],
                                      SEMS[b1]).wait()
                pltpu.make_async_copy(WV[b1].at[pl.ds(L, CHUNK)],
                                      den_sh.at[SDV[qp].at[1]],
                                      SEMS[b1]).wait()
            @pl.when(k + 1 < RPT)
            def _():
                pltpu.make_async_copy(sd_hbm.at[wid, k + 1], SDV[qn],
                                      SEMI[qn]).wait()
                pltpu.async_copy(h_hbm.at[SDV[qn].at[0]], ROWS[b1], SEMG[b1])
            @pl.when(k + 2 < RPT)
            def _():
                pltpu.async_copy(sd_hbm.at[wid, k + 2], SDV[qf], SEMI[qf])
            for v in range(CHUNK // L):
                si = SDV[qi][0, pl.ds(v * L, L)]
                di = SDV[qi][1, pl.ds(v * L, L)]
                a = plsc.load_gather(asv, [si]) + plsc.load_gather(adv, [di])
                e = jnp.where(a > 0, a, 0.2 * a)
                WV[b][pl.ds(L + v * L, L)] = jnp.exp(e)
            wvb = WV[b]
            rowsb = ROWS[b]

            def _scale_grp(g, _):
                rg = g * 8
                for r in range(8):
                    wr_idx = jnp.zeros((L,), jnp.int32) + (rg + (L + r))
                    wr = plsc.load_gather(wvb, [wr_idx])
                    for u in range(D // L):
                        rowsb[rg + r, pl.ds(u * L, L)] = (
                            rowsb[rg + r, pl.ds(u * L, L)] * wr)
                return ()
            lax.fori_loop(0, CHUNK // 8, _scale_grp, ())
            pltpu.async_copy(ROWS[b], num_sh.at[SDV[qi].at[1]], SEMS[b],
                             add=True)
            pltpu.async_copy(WV[b].at[pl.ds(L, CHUNK)],
                             den_sh.at[SDV[qi].at[1]], SEMS[b], add=True)
        return ()

    lax.fori_loop(0, RPT // 4, _super, ())

    pltpu.make_async_copy(rows1, num_sh.at[sdv3.at[1]], semS1).wait()
    pltpu.make_async_copy(wv1.at[pl.ds(L, CHUNK)], den_sh.at[sdv3.at[1]],
                          semS1).wait()
    plsc.subcore_barrier()

    pltpu.sync_copy(num_sh.at[pl.ds(base, ROWS_PER_SUB)],
                    num_out.at[c, pl.ds(base, ROWS_PER_SUB)])
    pltpu.sync_copy(den_sh.at[pl.ds(base, ROWS_PER_SUB)],
                    zv.at[pl.ds(0, ROWS_PER_SUB)])
    pltpu.sync_copy(zv.at[pl.ds(0, ROWS_PER_SUB)],
                    den_out.at[pl.ds(c * NP + base, ROWS_PER_SUB)])


def kernel(x, edge_index, W1, att_src1, att_dst1, b1, W2, att_src2,
           att_dst2, b2):
    loop = jnp.arange(N, dtype=jnp.int32)
    pad = PAD_E - E_TOT
    padi = jnp.arange(pad, dtype=jnp.int32)
    src = jnp.concatenate([edge_index[0], loop, padi % N])
    dst = jnp.concatenate([edge_index[1], loop, N + padi % (NP - N)])
    sd = jnp.stack([src.reshape(NW, RPT, CHUNK),
                    dst.reshape(NW, RPT, CHUNK)], axis=2)
    x_pad = jnp.zeros((NP, D), jnp.float32).at[:N].set(x)
    att2_1 = jnp.stack([att_src1, att_dst1], axis=1)
    att2_2 = jnp.stack([att_src2, att_dst2], axis=1)

    h1, a2_1 = _tc_feat(x_pad, W1, att2_1)
    asrc1 = a2_1[:, 0].at[N:].set(-1e30)
    adst1 = a2_1[:, 1].at[N:].set(-1e30)
    num1, den1 = _sc_edge(sd, asrc1, adst1, h1)
    den1 = den1.reshape(NC, NP)

    h2, a2_2 = _tc_mid(num1, den1, b1.reshape(1, D), W2, att2_2)
    asrc2 = a2_2[:, 0].at[N:].set(-1e30)
    adst2 = a2_2[:, 1].at[N:].set(-1e30)
    num2, den2 = _sc_edge(sd, asrc2, adst2, h2)
    den2 = den2.reshape(NC, NP)

    h2b = _tc_fin(num2, den2, b2.reshape(1, D))
    return _tc_mm(h2b)


# speedup vs baseline: 3.1026x; 1.0649x over previous
import functools

import jax
import jax.numpy as jnp
from jax import lax
from jax.experimental import pallas as pl
from jax.experimental.pallas import tpu as pltpu
from jax.experimental.pallas import tpu_sc as plsc

N = 10000
D = 128
E = 320000
E_TOT = E + N
NC, NS, L = 2, 16, 16
NW = NC * NS
CHUNK = 48
RPT = 216
PAD_E = NW * RPT * CHUNK
NP = 10112
ROWS_PER_SUB = NP // NS


def _tc_feat_body(x_ref, w_ref, att2_ref, h_ref, a2_ref):
    h = lax.dot_general(x_ref[...], w_ref[...], (((1,), (1,)), ((), ())),
                        preferred_element_type=jnp.float32)
    h_ref[...] = h
    a2_ref[...] = lax.dot_general(h, att2_ref[...], (((1,), (0,)), ((), ())),
                                  preferred_element_type=jnp.float32)


def _tc_feat(x_pad, w, att2):
    return pl.pallas_call(
        _tc_feat_body,
        out_shape=(jax.ShapeDtypeStruct((NP, D), jnp.float32),
                   jax.ShapeDtypeStruct((NP, 2), jnp.float32)),
    )(x_pad, w, att2)


def _tc_mid_body(num_ref, den_ref, b_ref, w_ref, att2_ref, h_ref, a2_ref):
    den = den_ref[0, :] + den_ref[1, :]
    o = (num_ref[0] + num_ref[1]) / (den[:, None] + 1e-16) + b_ref[...]
    o = jnp.where(o > 0, o, 0.02 * o)
    h = lax.dot_general(o, w_ref[...], (((1,), (1,)), ((), ())),
                        preferred_element_type=jnp.float32)
    h_ref[...] = h
    a2_ref[...] = lax.dot_general(h, att2_ref[...], (((1,), (0,)), ((), ())),
                                  preferred_element_type=jnp.float32)


def _tc_mid(num, den, b, w, att2):
    return pl.pallas_call(
        _tc_mid_body,
        out_shape=(jax.ShapeDtypeStruct((NP, D), jnp.float32),
                   jax.ShapeDtypeStruct((NP, 2), jnp.float32)),
    )(num, den, b, w, att2)


def _tc_fin_body(num_ref, den_ref, b_ref, h_ref):
    den = den_ref[0, :] + den_ref[1, :]
    o = (num_ref[0] + num_ref[1]) / (den[:, None] + 1e-16) + b_ref[...]
    o = jnp.where(o > 0, o, 0.02 * o)
    h_ref[...] = o[:N, :]


def _tc_fin(num, den, b):
    return pl.pallas_call(
        _tc_fin_body,
        out_shape=jax.ShapeDtypeStruct((N, D), jnp.float32),
    )(num, den, b)


BM = 400


def _tc_mm_body(a_ref, b_ref, o_ref):
    o_ref[...] = lax.dot_general(a_ref[...], b_ref[...],
                                 (((1,), (1,)), ((), ())),
                                 preferred_element_type=jnp.float32)


def _tc_mm(h):
    return pl.pallas_call(
        _tc_mm_body,
        grid=(N // BM,),
        in_specs=[pl.BlockSpec((BM, D), lambda i: (i, 0)),
                  pl.BlockSpec((N, D), lambda i: (0, 0))],
        out_specs=pl.BlockSpec((BM, N), lambda i: (i, 0)),
        out_shape=jax.ShapeDtypeStruct((N, N), jnp.float32),
    )(h, h)


_MESH = plsc.VectorSubcoreMesh(core_axis_name="c", subcore_axis_name="s",
                               num_cores=NC, num_subcores=NS)


@functools.partial(
    pl.kernel,
    out_type=(jax.ShapeDtypeStruct((NC, NP, D), jnp.float32),
              jax.ShapeDtypeStruct((NC * NP,), jnp.float32)),
    mesh=_MESH,
    compiler_params=pltpu.CompilerParams(needs_layout_passes=False),
    scratch_types=(
        [pltpu.VMEM((2, CHUNK), jnp.int32)] * 8 +
        [pltpu.VMEM((CHUNK, D), jnp.float32)] * 4 +
        [pltpu.VMEM((NP,), jnp.float32),
         pltpu.VMEM((NP,), jnp.float32)] +
        [pltpu.VMEM((CHUNK + L,), jnp.float32)] * 4 +
        [pltpu.VMEM((640,), jnp.float32),
         pltpu.MemorySpace.VMEM_SHARED((NP, D), jnp.float32),
         pltpu.MemorySpace.VMEM_SHARED((NP,), jnp.float32)] +
        [pltpu.SemaphoreType.DMA] * 16
    ),
)
def _sc_edge(sd_hbm, asrc_hbm, adst_hbm, h_hbm, num_out, den_out,
             sdv0, sdv1, sdv2, sdv3, sdv4, sdv5, sdv6, sdv7,
             rows0, rows1, rows2, rows3, asv, adv,
             wv0, wv1, wv2, wv3, zv, num_sh, den_sh,
             semI0, semI1, semI2, semI3, semI4, semI5, semI6, semI7,
             semG0, semG1, semG2, semG3, semS0, semS1, semS2, semS3):
    SDV = [sdv0, sdv1, sdv2, sdv3, sdv4, sdv5, sdv6, sdv7]
    ROWS = [rows0, rows1, rows2, rows3]
    WV = [wv0, wv1, wv2, wv3]
    SEMI = [semI0, semI1, semI2, semI3, semI4, semI5, semI6, semI7]
    SEMG = [semG0, semG1, semG2, semG3]
    SEMS = [semS0, semS1, semS2, semS3]

    c = lax.axis_index("c")
    s = lax.axis_index("s")
    wid = c * NS + s

    zero16 = jnp.zeros((L,), jnp.float32)

    def _z(i, _):
        zv[pl.ds(i * L, L)] = zero16
        return ()
    lax.fori_loop(0, 640 // L, _z, ())
    base = s * ROWS_PER_SUB
    pltpu.sync_copy(zv.at[pl.ds(0, ROWS_PER_SUB)],
                    den_sh.at[pl.ds(base, ROWS_PER_SUB)])

    def _zrows(i, _):
        def _zcol(u, _):
            rows0[i, pl.ds(u * L, L)] = zero16
            return ()
        lax.fori_loop(0, D // L, _zcol, ())
        return ()
    lax.fori_loop(0, CHUNK, _zrows, ())
    nz = (ROWS_PER_SUB + CHUNK - 1) // CHUNK
    for k in range(nz):
        sz = min(CHUNK, ROWS_PER_SUB - k * CHUNK)
        pltpu.sync_copy(rows0.at[pl.ds(0, sz)],
                        num_sh.at[pl.ds(base + k * CHUNK, sz)])
    plsc.subcore_barrier()

    pltpu.sync_copy(asrc_hbm, asv)
    pltpu.sync_copy(adst_hbm, adv)
    for p in range(4):
        pltpu.async_copy(sd_hbm.at[wid, p], SDV[p], SEMI[p])
    for p in range(2):
        pltpu.make_async_copy(sd_hbm.at[wid, p], SDV[p], SEMI[p]).wait()
        pltpu.async_copy(h_hbm.at[SDV[p].at[0]], ROWS[p], SEMG[p])

    def _super(jj, _):
        for q in range(8):
            k = jj * 8 + q
            t = q % 4
            td = (q + 2) % 4
            tg = (q + 2) % 4
            u = q
            ud = (q + 6) % 8
            ug = (q + 2) % 8
            uf = (q + 4) % 8
            pltpu.make_async_copy(h_hbm.at[SDV[u].at[0]], ROWS[t],
                                  SEMG[t]).wait()
            @pl.when(k >= 2)
            def _():
                pltpu.make_async_copy(ROWS[td], num_sh.at[SDV[ud].at[1]],
                                      SEMS[td]).wait()
                pltpu.make_async_copy(WV[td].at[pl.ds(L, CHUNK)],
                                      den_sh.at[SDV[ud].at[1]],
                                      SEMS[td]).wait()
            @pl.when(k + 2 < RPT)
            def _():
                pltpu.make_async_copy(sd_hbm.at[wid, k + 2], SDV[ug],
                                      SEMI[ug]).wait()
                pltpu.async_copy(h_hbm.at[SDV[ug].at[0]], ROWS[tg], SEMG[tg])
            @pl.when(k + 4 < RPT)
            def _():
                pltpu.async_copy(sd_hbm.at[wid, k + 4], SDV[uf], SEMI[uf])
            for v in range(CHUNK // L):
                si = SDV[u][0, pl.ds(v * L, L)]
                di = SDV[u][1, pl.ds(v * L, L)]
                a = plsc.load_gather(asv, [si]) + plsc.load_gather(adv, [di])
                e = jnp.where(a > 0, a, 0.2 * a)
                WV[t][pl.ds(L + v * L, L)] = jnp.exp(e)
            wvb = WV[t]
            rowsb = ROWS[t]

            def _scale_grp(g, _):
                rg = g * 8
                for r in range(8):
                    wr_idx = jnp.zeros((L,), jnp.int32) + (rg + (L + r))
                    wr = plsc.load_gather(wvb, [wr_idx])
                    for uu in range(D // L):
                        rowsb[rg + r, pl.ds(uu * L, L)] = (
                            rowsb[rg + r, pl.ds(uu * L, L)] * wr)
                return ()
            lax.fori_loop(0, CHUNK // 8, _scale_grp, ())
            pltpu.async_copy(ROWS[t], num_sh.at[SDV[u].at[1]], SEMS[t],
                             add=True)
            pltpu.async_copy(WV[t].at[pl.ds(L, CHUNK)],
                             den_sh.at[SDV[u].at[1]], SEMS[t], add=True)
        return ()

    lax.fori_loop(0, RPT // 8, _super, ())

    for kk in (RPT - 2, RPT - 1):
        tt = kk % 4
        uu = kk % 8
        pltpu.make_async_copy(ROWS[tt], num_sh.at[SDV[uu].at[1]],
                              SEMS[tt]).wait()
        pltpu.make_async_copy(WV[tt].at[pl.ds(L, CHUNK)],
                              den_sh.at[SDV[uu].at[1]], SEMS[tt]).wait()
    plsc.subcore_barrier()

    pltpu.sync_copy(num_sh.at[pl.ds(base, ROWS_PER_SUB)],
                    num_out.at[c, pl.ds(base, ROWS_PER_SUB)])
    pltpu.sync_copy(den_sh.at[pl.ds(base, ROWS_PER_SUB)],
                    zv.at[pl.ds(0, ROWS_PER_SUB)])
    pltpu.sync_copy(zv.at[pl.ds(0, ROWS_PER_SUB)],
                    den_out.at[pl.ds(c * NP + base, ROWS_PER_SUB)])


def kernel(x, edge_index, W1, att_src1, att_dst1, b1, W2, att_src2,
           att_dst2, b2):
    loop = jnp.arange(N, dtype=jnp.int32)
    pad = PAD_E - E_TOT
    padi = jnp.arange(pad, dtype=jnp.int32)
    src = jnp.concatenate([edge_index[0], loop, padi % N])
    dst = jnp.concatenate([edge_index[1], loop, N + padi % (NP - N)])
    sd = jnp.stack([src.reshape(NW, RPT, CHUNK),
                    dst.reshape(NW, RPT, CHUNK)], axis=2)
    x_pad = jnp.zeros((NP, D), jnp.float32).at[:N].set(x)
    att2_1 = jnp.stack([att_src1, att_dst1], axis=1)
    att2_2 = jnp.stack([att_src2, att_dst2], axis=1)

    h1, a2_1 = _tc_feat(x_pad, W1, att2_1)
    asrc1 = a2_1[:, 0].at[N:].set(-1e30)
    adst1 = a2_1[:, 1].at[N:].set(-1e30)
    num1, den1 = _sc_edge(sd, asrc1, adst1, h1)
    den1 = den1.reshape(NC, NP)

    h2, a2_2 = _tc_mid(num1, den1, b1.reshape(1, D), W2, att2_2)
    asrc2 = a2_2[:, 0].at[N:].set(-1e30)
    adst2 = a2_2[:, 1].at[N:].set(-1e30)
    num2, den2 = _sc_edge(sd, asrc2, adst2, h2)
    den2 = den2.reshape(NC, NP)

    h2b = _tc_fin(num2, den2, b2.reshape(1, D))
    return _tc_mm(h2b)
